# Initial kernel scaffold; baseline (speedup 1.0000x reference)
import jax
import jax.numpy as jnp
from jax.experimental import pallas as pl


def kernel(x_enc, x_mark_enc, x_dec, x_mark_dec, params):
    raise NotImplementedError("write your pallas kernel here")


# speedup vs baseline: 1.0001x; 1.0001x over previous
import jax
import jax.numpy as jnp
from jax import lax
import numpy as np
from jax.experimental import pallas as pl

B = 2
SEQ_LEN = 2048
LABEL_LEN = 512
PRED_LEN = 512
DEC_LEN = LABEL_LEN + PRED_LEN
ENC_IN = 7
DEC_IN = 7
C_OUT = 7
D_MODEL = 768
N_HEADS = 12
D_FF = 1536
E_LAYERS = 3
D_LAYERS = 2
FACTOR = 5
MARK_DIM = 4
EPS_LN = 1e-5


def _pos_embedding(L, d_model):
    pos = np.arange(L, dtype=np.float32)[:, None]
    div = np.exp(np.arange(0, d_model, 2, dtype=np.float32) * -(np.log(10000.0) / d_model))
    pe = np.zeros((L, d_model), dtype=np.float32)
    pe[:, 0::2] = np.sin(pos * div)
    pe[:, 1::2] = np.cos(pos * div)
    return jnp.asarray(pe)


def _layer_norm(x, p):
    m = jnp.mean(x, axis=-1, keepdims=True)
    v = jnp.mean((x - m) ** 2, axis=-1, keepdims=True)
    return (x - m) / jnp.sqrt(v + EPS_LN) * p['g'] + p['b']


def _circ_conv1d(x, w, b=None):
    xt = jnp.transpose(x, (0, 2, 1))
    xt = jnp.concatenate([xt[:, :, -1:], xt, xt[:, :, :1]], axis=-1)
    out = lax.conv_general_dilated(xt, w, (1,), 'VALID', dimension_numbers=('NCH', 'OIH', 'NCH'))
    if b is not None:
        out = out + b[None, :, None]
    return jnp.transpose(out, (0, 2, 1))


def _prob_attention(q, k, v, mask_flag, seed):
    Bq, L_Q, H, D = q.shape
    L_K = k.shape[1]
    q = jnp.transpose(q, (0, 2, 1, 3))
    k = jnp.transpose(k, (0, 2, 1, 3))
    v = jnp.transpose(v, (0, 2, 1, 3))
    U_part = min(int(FACTOR * np.ceil(np.log(L_K))), L_K)
    u = min(int(FACTOR * np.ceil(np.log(L_Q))), L_Q)
    rng = np.random.default_rng(seed)
    index_sample = jnp.asarray(rng.integers(0, L_K, size=(L_Q, U_part)), dtype=jnp.int32)
    k_sample = k[:, :, index_sample, :]
    qk_sample = jnp.einsum('bhld,bhlsd->bhls', q, k_sample)
    M = qk_sample.max(axis=-1) - qk_sample.sum(axis=-1) / L_K
    _, m_top = lax.top_k(M, u)
    q_reduce = jnp.take_along_axis(q, m_top[..., None], axis=2)
    scores = jnp.einsum('bhud,bhkd->bhuk', q_reduce, k) * (1.0 / np.sqrt(D))
    if mask_flag:
        key_idx = jnp.arange(L_K)[None, None, None, :]
        causal = key_idx > m_top[..., None]
        scores = jnp.where(causal, -jnp.inf, scores)
        context = jnp.cumsum(v, axis=2)
    else:
        context = jnp.broadcast_to(v.mean(axis=2, keepdims=True), (Bq, H, L_Q, D))
    attn = jax.nn.softmax(scores, axis=-1)
    update = jnp.einsum('bhuk,bhkd->bhud', attn, v)
    bi = jnp.arange(Bq)[:, None, None]
    hi = jnp.arange(H)[None, :, None]
    context = context.at[bi, hi, m_top].set(update)
    return jnp.transpose(context, (0, 2, 1, 3))


def _attention_layer(p, x_q, x_kv, mask_flag, seed):
    Bq, L, _ = x_q.shape
    S = x_kv.shape[1]
    dh = D_MODEL // N_HEADS
    q = (x_q @ p['q']['w'].T + p['q']['b']).reshape(Bq, L, N_HEADS, dh)
    k = (x_kv @ p['k']['w'].T + p['k']['b']).reshape(Bq, S, N_HEADS, dh)
    v = (x_kv @ p['v']['w'].T + p['v']['b']).reshape(Bq, S, N_HEADS, dh)
    out = _prob_attention(q, k, v, mask_flag, seed).reshape(Bq, L, D_MODEL)
    return out @ p['o']['w'].T + p['o']['b']


def _ffn(x, p1, p2):
    y = x @ p1['w'].T + p1['b']
    y = jax.nn.gelu(y, approximate=False)
    return y @ p2['w'].T + p2['b']


def _encoder_layer(p, x, seed):
    x = x + _attention_layer(p['attn'], x, x, False, seed)
    x = _layer_norm(x, p['n1'])
    y = _ffn(x, p['c1'], p['c2'])
    return _layer_norm(x + y, p['n2'])


def _conv_layer(p, x):
    y = _circ_conv1d(x, p['w'], p['b'])
    y = y / jnp.sqrt(1.0 + 1e-5)
    y = jax.nn.elu(y)
    yt = jnp.transpose(y, (0, 2, 1))
    yt = lax.reduce_window(yt, -jnp.inf, lax.max, (1, 1, 3), (1, 1, 2), [(0, 0), (0, 0), (1, 1)])
    return jnp.transpose(yt, (0, 2, 1))


def _decoder_layer(p, x, cross, seed):
    x = x + _attention_layer(p['self'], x, x, True, seed)
    x = _layer_norm(x, p['n1'])
    x = x + _attention_layer(p['cross'], x, cross, False, seed + 1)
    x = _layer_norm(x, p['n2'])
    y = _ffn(x, p['c1'], p['c2'])
    return _layer_norm(x + y, p['n3'])


def _embed(x, x_mark, p):
    val = _circ_conv1d(x, p['token_w'])
    pos = _pos_embedding(x.shape[1], D_MODEL)[None]
    temp = x_mark @ p['temp_w'].T
    return val + pos + temp


def kernel(x_enc, x_mark_enc, x_dec, x_mark_dec, params):
    enc = _embed(x_enc, x_mark_enc, params['enc_emb'])
    for i in range(E_LAYERS - 1):
        enc = _encoder_layer(params['enc_layers'][i], enc, 1000 + i)
        enc = _conv_layer(params['enc_convs'][i], enc)
    enc = _encoder_layer(params['enc_layers'][E_LAYERS - 1], enc, 1000 + E_LAYERS - 1)
    enc = _layer_norm(enc, params['enc_norm'])
    dec = _embed(x_dec, x_mark_dec, params['dec_emb'])
    for i in range(D_LAYERS):
        dec = _decoder_layer(params['dec_layers'][i], dec, enc, 2000 + 10 * i)
    dec = _layer_norm(dec, params['dec_norm'])
    dec = dec @ params['proj']['w'].T + params['proj']['b']
    return dec[:, -PRED_LEN:, :]


# speedup vs baseline: 3.3184x; 3.3179x over previous
import functools

import numpy as np
import jax
import jax.numpy as jnp
from jax import lax
from jax.experimental import pallas as pl
from jax.experimental.pallas import tpu as pltpu

BB = 2
SEQ = 2048
PRED = 512
DECL = 1024
DM = 768
NH = 12
DH = 64
DFF = 1536
FACT = 5
EPS = 1e-5
TR = 512


@functools.lru_cache(maxsize=None)
def _amatT_np(seed, L_Q, L_K, U):
    rng = np.random.default_rng(seed)
    idx = rng.integers(0, L_K, size=(L_Q, U))
    A = np.zeros((L_K, L_Q), np.int8)
    np.add.at(A, (idx.ravel(), np.repeat(np.arange(L_Q), U)), 1)
    return A


@functools.lru_cache(maxsize=None)
def _pos_np(L):
    pos = np.arange(L, dtype=np.float32)[:, None]
    div = np.exp(np.arange(0, DM, 2, dtype=np.float32) * -(np.log(10000.0) / DM))
    pe = np.zeros((L, DM), dtype=np.float32)
    pe[:, 0::2] = np.sin(pos * div)
    pe[:, 1::2] = np.cos(pos * div)
    return np.tile(pe, (BB, 1))


def _bf(t):
    return t.astype(jnp.bfloat16)


def _dotbf(a, b):
    return jnp.dot(_bf(a), _bf(b), preferred_element_type=jnp.float32)


def _dg3(a, b):
    dims = (((1,), (1,)), ((), ()))
    ahi, bhi = _bf(a), _bf(b)
    alo = _bf(a - ahi.astype(jnp.float32))
    blo = _bf(b - bhi.astype(jnp.float32))
    d = lambda x, y: lax.dot_general(x, y, dims, preferred_element_type=jnp.float32)
    return d(ahi, bhi) + d(ahi, blo) + d(alo, bhi)


def _lnv(y, g, b):
    m = jnp.mean(y, axis=-1, keepdims=True)
    v = jnp.mean((y - m) ** 2, axis=-1, keepdims=True)
    return (y - m) / jnp.sqrt(v + EPS) * g + b


def _gelu(y):
    return 0.5 * y * (1.0 + lax.erf(y * np.float32(1.0 / np.sqrt(2.0))))


def _linear_body(x_ref, w_ref, b_ref, o_ref):
    o_ref[...] = _dotbf(x_ref[...], w_ref[...]) + b_ref[...]


def _linear(x, wT, b):
    R, K = x.shape
    N = wT.shape[1]
    return pl.pallas_call(
        _linear_body,
        grid=(R // TR,),
        in_specs=[pl.BlockSpec((TR, K), lambda i: (i, 0)),
                  pl.BlockSpec((K, N), lambda i: (0, 0)),
                  pl.BlockSpec((1, N), lambda i: (0, 0))],
        out_specs=pl.BlockSpec((TR, N), lambda i: (i, 0)),
        out_shape=jax.ShapeDtypeStruct((R, N), jnp.float32),
    )(x, wT, b)


def _projresln_body(c_ref, r_ref, w_ref, b_ref, g_ref, bb_ref, o_ref):
    y = r_ref[...] + _dotbf(c_ref[...], w_ref[...]) + b_ref[...]
    o_ref[...] = _lnv(y, g_ref[...], bb_ref[...])


def _projresln(ctx, res, p, pn):
    R = ctx.shape[0]
    return pl.pallas_call(
        _projresln_body,
        grid=(R // TR,),
        in_specs=[pl.BlockSpec((TR, DM), lambda i: (i, 0)),
                  pl.BlockSpec((TR, DM), lambda i: (i, 0)),
                  pl.BlockSpec((DM, DM), lambda i: (0, 0)),
                  pl.BlockSpec((1, DM), lambda i: (0, 0)),
                  pl.BlockSpec((1, DM), lambda i: (0, 0)),
                  pl.BlockSpec((1, DM), lambda i: (0, 0))],
        out_specs=pl.BlockSpec((TR, DM), lambda i: (i, 0)),
        out_shape=jax.ShapeDtypeStruct((R, DM), jnp.float32),
    )(ctx, res, p['w'].T, p['b'][None], pn['g'][None], pn['b'][None])


def _ffnln_body(x_ref, w1_ref, b1_ref, w2_ref, b2_ref, g_ref, bb_ref, o_ref):
    x = x_ref[...]
    h = _gelu(_dotbf(x, w1_ref[...]) + b1_ref[...])
    y = x + _dotbf(h, w2_ref[...]) + b2_ref[...]
    o_ref[...] = _lnv(y, g_ref[...], bb_ref[...])


def _ffnln(x, p1, p2, pn):
    R = x.shape[0]
    return pl.pallas_call(
        _ffnln_body,
        grid=(R // TR,),
        in_specs=[pl.BlockSpec((TR, DM), lambda i: (i, 0)),
                  pl.BlockSpec((DM, DFF), lambda i: (0, 0)),
                  pl.BlockSpec((1, DFF), lambda i: (0, 0)),
                  pl.BlockSpec((DFF, DM), lambda i: (0, 0)),
                  pl.BlockSpec((1, DM), lambda i: (0, 0)),
                  pl.BlockSpec((1, DM), lambda i: (0, 0)),
                  pl.BlockSpec((1, DM), lambda i: (0, 0))],
        out_specs=pl.BlockSpec((TR, DM), lambda i: (i, 0)),
        out_shape=jax.ShapeDtypeStruct((R, DM), jnp.float32),
    )(x, p1['w'].T, p1['b'][None], p2['w'].T, p2['b'][None], pn['g'][None], pn['b'][None])


def _lnlinear_body(x_ref, g_ref, bb_ref, w_ref, b_ref, o_ref):
    xn = _lnv(x_ref[...], g_ref[...], bb_ref[...])
    o_ref[...] = _dotbf(xn, w_ref[...]) + b_ref[...]


def _lnlinear(x, pn, wT, b):
    R = x.shape[0]
    N = wT.shape[1]
    return pl.pallas_call(
        _lnlinear_body,
        grid=(R // TR,),
        in_specs=[pl.BlockSpec((TR, DM), lambda i: (i, 0)),
                  pl.BlockSpec((1, DM), lambda i: (0, 0)),
                  pl.BlockSpec((1, DM), lambda i: (0, 0)),
                  pl.BlockSpec((DM, N), lambda i: (0, 0)),
                  pl.BlockSpec((1, N), lambda i: (0, 0))],
        out_specs=pl.BlockSpec((TR, N), lambda i: (i, 0)),
        out_shape=jax.ShapeDtypeStruct((R, N), jnp.float32),
    )(x, pn['g'][None], pn['b'][None], wT, b)


def _embed_body(x_ref, w_ref, pos_ref, o_ref):
    o_ref[...] = _dotbf(x_ref[...], w_ref[...]) + pos_ref[...]


def _embed(x, x_mark, p, L):
    tok = p['token_w']
    wT = jnp.concatenate([tok[:, :, 0].T, tok[:, :, 1].T, tok[:, :, 2].T,
                          p['temp_w'].T], axis=0)
    xm1 = jnp.roll(x, 1, axis=1)
    xp1 = jnp.roll(x, -1, axis=1)
    K = wT.shape[0]
    xcat = jnp.concatenate([xm1, x, xp1, x_mark], axis=-1).reshape(BB * L, K)
    pos = jnp.asarray(_pos_np(L))
    R = BB * L
    return pl.pallas_call(
        _embed_body,
        grid=(R // TR,),
        in_specs=[pl.BlockSpec((TR, K), lambda i: (i, 0)),
                  pl.BlockSpec((K, DM), lambda i: (0, 0)),
                  pl.BlockSpec((TR, DM), lambda i: (i, 0))],
        out_specs=pl.BlockSpec((TR, DM), lambda i: (i, 0)),
        out_shape=jax.ShapeDtypeStruct((R, DM), jnp.float32),
    )(xcat, wT, pos)


def _make_conv_body(L):
    def body(x_ref, w0_ref, w1_ref, w2_ref, b_ref, o_ref):
        x = x_ref[0]
        xm1 = jnp.concatenate([x[L - 1:, :], x[:L - 1, :]], axis=0)
        xp1 = jnp.concatenate([x[1:, :], x[:1, :]], axis=0)
        y = (_dotbf(xm1, w0_ref[...]) + _dotbf(x, w1_ref[...])
             + _dotbf(xp1, w2_ref[...]) + b_ref[...])
        y = y * np.float32(1.0 / np.sqrt(1.0 + 1e-5))
        y = jnp.where(y > 0, y, jnp.exp(jnp.minimum(y, 0.0)) - 1.0)
        y2 = y.reshape(L // 2, 2, DM)
        a = y2[:, 0, :]
        bo = y2[:, 1, :]
        bprev = jnp.concatenate(
            [jnp.full((1, DM), -jnp.inf, jnp.float32), bo[:L // 2 - 1, :]], axis=0)
        o_ref[0] = jnp.maximum(jnp.maximum(bprev, a), bo)
    return body


def _conv_distil(x, p):
    _, L, _ = x.shape
    w = p['w']
    return pl.pallas_call(
        _make_conv_body(L),
        grid=(BB,),
        in_specs=[pl.BlockSpec((1, L, DM), lambda i: (i, 0, 0)),
                  pl.BlockSpec((DM, DM), lambda i: (0, 0)),
                  pl.BlockSpec((DM, DM), lambda i: (0, 0)),
                  pl.BlockSpec((DM, DM), lambda i: (0, 0)),
                  pl.BlockSpec((1, DM), lambda i: (0, 0))],
        out_specs=pl.BlockSpec((1, L // 2, DM), lambda i: (i, 0, 0)),
        out_shape=jax.ShapeDtypeStruct((BB, L // 2, DM), jnp.float32),
    )(x, w[:, :, 0].T, w[:, :, 1].T, w[:, :, 2].T, p['b'][None])


def _make_attn_body(L_Q, L_K, utop, causal):
    scale = np.float32(1.0 / np.sqrt(DH))
    KCH = min(L_K, 512)

    def body(q_ref, k_ref, v_ref, a_ref, o_ref):
        q = q_ref[0]
        k = k_ref[0]
        v = v_ref[0]
        Mmax = jnp.full((1, L_Q), -jnp.inf, jnp.float32)
        Msum = jnp.zeros((1, L_Q), jnp.float32)
        for c0 in range(0, L_K, KCH):
            kc = k[c0:c0 + KCH, :]
            STc = _dg3(kc, q)
            Ac = a_ref[c0:c0 + KCH, :].astype(jnp.float32)
            Mmax = jnp.maximum(
                Mmax,
                jnp.max(jnp.where(Ac > 0.0, STc, -jnp.inf), axis=0, keepdims=True))
            Msum = Msum + jnp.sum(Ac * STc, axis=0, keepdims=True)
        M = Mmax - Msum * np.float32(1.0 / L_K)
        lane = lax.broadcasted_iota(jnp.int32, (1, L_Q), 1)
        riota = lax.broadcasted_iota(jnp.int32, (utop, 1), 0)
        ciota = lax.broadcasted_iota(jnp.int32, (1, utop), 1)

        def tk_body(i, carry):
            Mc, svc, svr = carry
            mx = jnp.max(Mc)
            s = jnp.min(jnp.where(Mc == mx, lane, L_Q))
            svc = jnp.where(riota == i, s, svc)
            svr = jnp.where(ciota == i, s, svr)
            return jnp.where(lane == s, -jnp.inf, Mc), svc, svr

        _, selcol, selrow = lax.fori_loop(
            0, utop, tk_body,
            (M, jnp.zeros((utop, 1), jnp.int32), jnp.zeros((1, utop), jnp.int32)))
        if causal:
            ri = lax.broadcasted_iota(jnp.int32, (L_Q, L_K), 0)
            ci = lax.broadcasted_iota(jnp.int32, (L_Q, L_K), 1)
            tri = (ri >= ci).astype(jnp.float32)
            base = jnp.dot(tri, v, preferred_element_type=jnp.float32)
        else:
            vm = jnp.sum(v, axis=0, keepdims=True) * np.float32(1.0 / L_K)
            base = jnp.broadcast_to(vm, (L_Q, DH))
        liota = lax.broadcasted_iota(jnp.int32, (utop, L_Q), 1)
        onehot = (liota == selcol).astype(jnp.float32)
        qsel = (jnp.dot(_bf(onehot), _bf(q), preferred_element_type=jnp.float32)
                + jnp.dot(_bf(onehot), _bf(q - _bf(q).astype(jnp.float32)),
                          preferred_element_type=jnp.float32))
        sc = _dg3(qsel, k) * scale
        if causal:
            ck = lax.broadcasted_iota(jnp.int32, (utop, L_K), 1)
            sc = jnp.where(ck > selcol, -jnp.inf, sc)
        mx = jnp.max(sc, axis=1, keepdims=True)
        e = jnp.exp(sc - mx)
        patt = e / jnp.sum(e, axis=1, keepdims=True)
        upd = _dotbf(patt, v)
        ohT = (lax.broadcasted_iota(jnp.int32, (L_Q, utop), 0)
               == selrow).astype(jnp.float32)
        scat = (jnp.dot(_bf(ohT), _bf(upd), preferred_element_type=jnp.float32)
                + jnp.dot(_bf(ohT), _bf(upd - _bf(upd).astype(jnp.float32)),
                          preferred_element_type=jnp.float32))
        hit = jnp.sum(ohT, axis=1, keepdims=True) > 0.0
        o_ref[0] = jnp.where(hit, scat, base)

    return body


def _prob_attn(q4, k4, v4, seed, causal):
    Bq, H, L_Q, _ = q4.shape
    L_K = k4.shape[2]
    U = min(int(FACT * np.ceil(np.log(L_K))), L_K)
    utop = min(int(FACT * np.ceil(np.log(L_Q))), L_Q)
    amatT = jnp.asarray(_amatT_np(seed, L_Q, L_K, U))
    BH = Bq * H
    out = pl.pallas_call(
        _make_attn_body(L_Q, L_K, utop, causal),
        grid=(BH,),
        in_specs=[pl.BlockSpec((1, L_Q, DH), lambda i: (i, 0, 0)),
                  pl.BlockSpec((1, L_K, DH), lambda i: (i, 0, 0)),
                  pl.BlockSpec((1, L_K, DH), lambda i: (i, 0, 0)),
                  pl.BlockSpec((L_K, L_Q), lambda i: (0, 0))],
        out_specs=pl.BlockSpec((1, L_Q, DH), lambda i: (i, 0, 0)),
        out_shape=jax.ShapeDtypeStruct((BH, L_Q, DH), jnp.float32),
    )(q4.reshape(BH, L_Q, DH), k4.reshape(BH, L_K, DH), v4.reshape(BH, L_K, DH),
      amatT)
    return out.reshape(Bq, H, L_Q, DH)


def _self_attn_ctx(p, x3, causal, seed):
    Bq, L, _ = x3.shape
    wT = jnp.concatenate([p['q']['w'].T, p['k']['w'].T, p['v']['w'].T], axis=1)
    bb = jnp.concatenate([p['q']['b'], p['k']['b'], p['v']['b']])[None]
    qkv = _linear(x3.reshape(Bq * L, DM), wT, bb)
    qkv = qkv.reshape(Bq, L, 3, NH, DH).transpose(2, 0, 3, 1, 4)
    ctx = _prob_attn(qkv[0], qkv[1], qkv[2], seed, causal)
    return ctx.transpose(0, 2, 1, 3).reshape(Bq * L, DM)


def _cross_attn_ctx(p, x3, enc, enc_norm_p, seed):
    Bq, L, _ = x3.shape
    S = enc.shape[1]
    q = _linear(x3.reshape(Bq * L, DM), p['q']['w'].T, p['q']['b'][None])
    q4 = q.reshape(Bq, L, NH, DH).transpose(0, 2, 1, 3)
    wT = jnp.concatenate([p['k']['w'].T, p['v']['w'].T], axis=1)
    bb = jnp.concatenate([p['k']['b'], p['v']['b']])[None]
    kv = _lnlinear(enc.reshape(Bq * S, DM), enc_norm_p, wT, bb)
    kv = kv.reshape(Bq, S, 2, NH, DH).transpose(2, 0, 3, 1, 4)
    ctx = _prob_attn(q4, kv[0], kv[1], seed, False)
    return ctx.transpose(0, 2, 1, 3).reshape(Bq * L, DM)


def _enc_layer(p, x3, seed):
    Bq, L, _ = x3.shape
    ctx = _self_attn_ctx(p['attn'], x3, False, seed)
    x1 = _projresln(ctx, x3.reshape(-1, DM), p['attn']['o'], p['n1'])
    out = _ffnln(x1, p['c1'], p['c2'], p['n2'])
    return out.reshape(Bq, L, DM)


def _dec_layer(p, x3, enc, enc_norm_p, seed):
    Bq, L, _ = x3.shape
    ctx = _self_attn_ctx(p['self'], x3, True, seed)
    x1 = _projresln(ctx, x3.reshape(-1, DM), p['self']['o'], p['n1'])
    ctx2 = _cross_attn_ctx(p['cross'], x1.reshape(Bq, L, DM), enc, enc_norm_p,
                           seed + 1)
    x2 = _projresln(ctx2, x1, p['cross']['o'], p['n2'])
    out = _ffnln(x2, p['c1'], p['c2'], p['n3'])
    return out.reshape(Bq, L, DM)


def kernel(x_enc, x_mark_enc, x_dec, x_mark_dec, params):
    enc = _embed(x_enc, x_mark_enc, params['enc_emb'], SEQ).reshape(BB, SEQ, DM)
    enc = _enc_layer(params['enc_layers'][0], enc, 1000)
    enc = _conv_distil(enc, params['enc_convs'][0])
    enc = _enc_layer(params['enc_layers'][1], enc, 1001)
    enc = _conv_distil(enc, params['enc_convs'][1])
    enc = _enc_layer(params['enc_layers'][2], enc, 1002)
    dec = _embed(x_dec, x_mark_dec, params['dec_emb'], DECL).reshape(BB, DECL, DM)
    dec = _dec_layer(params['dec_layers'][0], dec, enc, params['enc_norm'], 2000)
    dec = _dec_layer(params['dec_layers'][1], dec, enc, params['enc_norm'], 2010)
    out = _lnlinear(dec.reshape(BB * DECL, DM), params['dec_norm'],
                    params['proj']['w'].T, params['proj']['b'][None])
    return out.reshape(BB, DECL, -1)[:, -PRED:, :]
